# Initial kernel scaffold; baseline (speedup 1.0000x reference)
import jax
import jax.numpy as jnp
from jax.experimental import pallas as pl


def kernel(points, g0_z_W, g0_z_b, g0_r_W, g0_r_b, g0_s_W, g0_s_b, g0_fc_W, g0_fc_b, g1_z_W, g1_z_b, g1_r_W, g1_r_b, g1_s_W, g1_s_b, g1_fc_W, g1_fc_b, pos_W, pos_b, t0_qkv_W, t0_out_W, t0_out_b, t0_ff1_W, t0_ff1_b, t0_ff2_W, t0_ff2_b, t0_ln1_g, t0_ln1_b, t0_ln2_g, t0_ln2_b, t1_qkv_W, t1_out_W, t1_out_b, t1_ff1_W, t1_ff1_b, t1_ff2_W, t1_ff2_b, t1_ln1_g, t1_ln1_b, t1_ln2_g, t1_ln2_b, head_fc1_W, head_fc1_b, head_fc2_W, head_fc2_b, head_ln_g, head_ln_b):
    raise NotImplementedError("write your pallas kernel here")


# speedup vs baseline: 18.5009x; 18.5009x over previous
import functools

import jax
import jax.numpy as jnp
from jax import lax
from jax.experimental import pallas as pl
from jax.experimental.pallas import tpu as pltpu
from jax.experimental.pallas import tpu_sc as plsc

B, T, N = 2, 4, 4096
S0, S1 = 1024, 256
C0, C1 = 64, 128
K0, KG, K1 = 64, 16, 48
R0SQ, RGSQ, R1SQ = 0.25, 0.25, 1.0
HEADS, DIM_HEAD, DEPTH, MLP_DIM = 4, 32, 2, 256
G = B * T
NEG = -3.4e38


def _fps_body(px, py, pz, x0, y0, z0, x1, y1, z1):
    def run_level(X, Y, Z, n, npt):
        iota = lax.broadcasted_iota(jnp.int32, (G, n), 1)
        iotap = lax.broadcasted_iota(jnp.int32, (G, npt), 1)

        def step(i, carry):
            dist, far, ax, ay, az = carry
            oh = (iota == far).astype(jnp.float32)
            cx = jnp.sum(X * oh, 1, keepdims=True)
            cy = jnp.sum(Y * oh, 1, keepdims=True)
            cz = jnp.sum(Z * oh, 1, keepdims=True)
            sel = (iotap == i).astype(jnp.float32)
            ax = ax + cx * sel
            ay = ay + cy * sel
            az = az + cz * sel
            dx = X - cx
            dy = Y - cy
            dz = Z - cz
            d = (dx * dx + dy * dy) + dz * dz
            dist = jnp.minimum(dist, d)
            m = jnp.max(dist, 1, keepdims=True)
            far = jnp.min(jnp.where(dist == m, iota, n), 1, keepdims=True)
            return dist, far, ax, ay, az

        zer = jnp.zeros((G, npt), jnp.float32)
        _, _, ax, ay, az = lax.fori_loop(
            0, npt, step,
            (jnp.full((G, n), 1e10, jnp.float32),
             jnp.zeros((G, 1), jnp.int32), zer, zer, zer))
        return ax, ay, az

    ax0, ay0, az0 = run_level(px[...], py[...], pz[...], N, S0)
    x0[...] = ax0
    y0[...] = ay0
    z0[...] = az0
    ax1, ay1, az1 = run_level(ax0, ay0, az0, S0, S1)
    x1[...] = ax1
    y1[...] = ay1
    z1[...] = az1


def _fps(px, py, pz, interpret=False):
    f32 = jnp.float32
    return pl.pallas_call(
        _fps_body,
        out_shape=[jax.ShapeDtypeStruct((G, S0), f32)] * 3
        + [jax.ShapeDtypeStruct((G, S1), f32)] * 3,
        interpret=interpret,
    )(px, py, pz)


def _take(v, idxvec):
    dn = lax.GatherDimensionNumbers(
        offset_dims=(), collapsed_slice_dims=(0,), start_index_map=(0,))
    return lax.gather(v, idxvec[:, None], dn, (1,),
                      mode=lax.GatherScatterMode.PROMISE_IN_BOUNDS)


def _make_sc_gm(nplanes, qsplit, s, ns, k, ch, r2, ppb):
    qpw = s // qsplit
    nch = ch // 16
    kg = k // 16
    nsc = ns // 16
    mesh = plsc.VectorSubcoreMesh(core_axis_name="c", subcore_axis_name="s",
                                  num_cores=2, num_subcores=16)

    def body(qx_h, qy_h, qz_h, sx_h, sy_h, sz_h, h_h, o_h,
             sxv, syv, szv, qxv, qyv, qzv, hv, ibuf, outv):
        wid = lax.axis_index("s") * 2 + lax.axis_index("c")
        plane = wid // qsplit
        qc = lax.rem(wid, qsplit)
        b = plane // ppb
        pltpu.sync_copy(sx_h.at[b], sxv)
        pltpu.sync_copy(sy_h.at[b], syv)
        pltpu.sync_copy(sz_h.at[b], szv)
        qoff = pl.multiple_of(qc * qpw, 16)
        pltpu.sync_copy(qx_h.at[b, pl.ds(qoff, qpw)], qxv)
        pltpu.sync_copy(qy_h.at[b, pl.ds(qoff, qpw)], qyv)
        pltpu.sync_copy(qz_h.at[b, pl.ds(qoff, qpw)], qzv)
        pltpu.sync_copy(h_h.at[plane], hv)
        iota16 = lax.iota(jnp.int32, 16)
        zeros16 = iota16 * 0

        def per_query(q, carry):
            qg = pl.multiple_of((q // 16) * 16, 16)
            lsel = zeros16 + lax.rem(q, 16)
            qxs = _take(qxv[pl.ds(qg, 16)], lsel)
            qys = _take(qyv[pl.ds(qg, 16)], lsel)
            qzs = _take(qzv[pl.ds(qg, 16)], lsel)
            ibuf[pl.ds(0, 16)] = zeros16

            def scan_step(cnt, ci):
                co = pl.multiple_of(ci * 16, 16)
                dx = sxv[pl.ds(co, 16)] - qxs
                dy = syv[pl.ds(co, 16)] - qys
                dz = szv[pl.ds(co, 16)] - qzs
                d = (dx * dx + dy * dy) + dz * dz
                m = d <= r2
                inc = plsc.cumsum(m.astype(jnp.int32))
                eff = m & (inc <= (k - cnt))
                plsc.store_scatter(ibuf, [cnt + inc - 1], iota16 + ci * 16,
                                   mask=eff)
                return cnt + jnp.sum(jnp.where(eff, 1, 0))

            def scan_chunk(ci, cnt):
                return lax.cond(cnt < k, lambda: scan_step(cnt, ci),
                                lambda: cnt)

            cnt = lax.fori_loop(0, nsc, scan_chunk, 0)
            cnt_v = zeros16 + cnt
            j0 = _take(ibuf[pl.ds(0, 16)], zeros16)

            def ggroup(g, acc):
                go = pl.multiple_of(g * 16, 16)
                jv = ibuf[pl.ds(go, 16)]
                valid = (iota16 + g * 16) < cnt_v
                jv = jnp.where(valid, jv, j0)
                for l in range(16):
                    js = _take(jv, zeros16 + l)
                    base = js * ch + iota16
                    acc = tuple(
                        jnp.maximum(acc[cc],
                                    plsc.load_gather(hv, [base + cc * 16]))
                        for cc in range(nch))
                return acc

            acc = lax.fori_loop(0, kg, ggroup,
                                tuple(zeros16.astype(jnp.float32) + NEG
                                      for _ in range(nch)))
            for cc in range(nch):
                outv[q, pl.ds(cc * 16, 16)] = acc[cc]
            return carry

        lax.fori_loop(0, qpw, per_query, 0)
        pltpu.sync_copy(outv, o_h.at[plane, pl.ds(qoff, qpw)])

    f32 = jnp.float32
    return pl.kernel(
        body,
        out_type=jax.ShapeDtypeStruct((nplanes, s, ch), f32),
        mesh=mesh,
        compiler_params=pltpu.CompilerParams(needs_layout_passes=False),
        scratch_types=[
            pltpu.VMEM((ns,), f32), pltpu.VMEM((ns,), f32),
            pltpu.VMEM((ns,), f32),
            pltpu.VMEM((qpw,), f32), pltpu.VMEM((qpw,), f32),
            pltpu.VMEM((qpw,), f32),
            pltpu.VMEM((ns * ch,), f32),
            pltpu.VMEM((k + 16,), jnp.int32),
            pltpu.VMEM((qpw, ch), f32),
        ],
    )


_sc_gm_cached = functools.lru_cache(None)(_make_sc_gm)


def _sc_g0(*a):
    return _sc_gm_cached(4, 8, S0, S0, K0, 96, R0SQ, 2)(*a)


def _sc_grp(*a):
    return _sc_gm_cached(2, 16, S1, S0, KG, C0, RGSQ, 1)(*a)


def _sc_g1(*a):
    return _sc_gm_cached(2, 16, S1, S1, K1, 384, R1SQ, 1)(*a)


def _outer3(x, y, z, wt):
    return (x[:, None] * wt[0][None, :] + y[:, None] * wt[1][None, :]
            + z[:, None] * wt[2][None, :])


def _h0_body(p2x, p2y, p2z, s2, p1x, p1y, p1z, wrelt, wst, bs, hh, off):
    w = wrelt[...]
    h = _outer3(p2x[0, 0], p2y[0, 0], p2z[0, 0], w) + jnp.dot(
        s2[0], wst[...], preferred_element_type=jnp.float32)
    hh[0, 0] = h[:, :96]
    hh[0, 1] = h[:, 96:]
    off[0] = bs[...] - _outer3(p1x[0, 0], p1y[0, 0], p1z[0, 0], w)


def _h0(p2x, p2y, p2z, s2, p1x, p1y, p1z, wrelt, wst, bs, interpret=False):
    f32 = jnp.float32
    sp = lambda *shp: pl.BlockSpec((1,) + shp, lambda b: (b,) + (0,) * len(shp))
    wsp = lambda a: pl.BlockSpec(a.shape, lambda b: (0,) * a.ndim)
    p2x, p2y, p2z = p2x[:, None], p2y[:, None], p2z[:, None]
    p1x, p1y, p1z = p1x[:, None], p1y[:, None], p1z[:, None]
    return pl.pallas_call(
        _h0_body,
        grid=(B,),
        in_specs=[sp(1, S0), sp(1, S0), sp(1, S0), sp(S0, C0), sp(1, S0),
                  sp(1, S0), sp(1, S0), wsp(wrelt), wsp(wst), wsp(bs)],
        out_specs=[sp(2, S0, 96), sp(S0, 192)],
        out_shape=[jax.ShapeDtypeStruct((B, 2, S0, 96), f32),
                   jax.ShapeDtypeStruct((B, S0, 192), f32)],
        interpret=interpret,
    )(p2x, p2y, p2z, s2, p1x, p1y, p1z, wrelt, wst, bs)


def _gru0_body(gmh, off0, fcwt, fcb, p2x, p2y, p2z, s1p, p1x, p1y, p1z,
               wrelt, wst, bs, s0n, h1, off1):
    o = jnp.concatenate([gmh[0, 0], gmh[0, 1]], axis=1) + off0[0]
    z = jax.nn.sigmoid(o[:, :C0])
    r = jax.nn.sigmoid(o[:, C0:2 * C0])
    so = o[:, 2 * C0:]
    sn = jnp.tanh(jnp.dot(r * so, fcwt[...],
                          preferred_element_type=jnp.float32) + fcb[...])
    s0n[0] = z * so + (1 - z) * sn
    w = wrelt[...]
    h1[0] = _outer3(p2x[0, 0], p2y[0, 0], p2z[0, 0], w) + jnp.dot(
        s1p[0], wst[...], preferred_element_type=jnp.float32)
    off1[0] = bs[...] - _outer3(p1x[0, 0], p1y[0, 0], p1z[0, 0], w)


def _gru0(gmh, off0, fcwt, fcb, p2x, p2y, p2z, s1p, p1x, p1y, p1z,
          wrelt, wst, bs, interpret=False):
    f32 = jnp.float32
    sp = lambda *shp: pl.BlockSpec((1,) + shp, lambda b: (b,) + (0,) * len(shp))
    wsp = lambda a: pl.BlockSpec(a.shape, lambda b: (0,) * a.ndim)
    p2x, p2y, p2z = p2x[:, None], p2y[:, None], p2z[:, None]
    p1x, p1y, p1z = p1x[:, None], p1y[:, None], p1z[:, None]
    return pl.pallas_call(
        _gru0_body,
        grid=(B,),
        in_specs=[sp(2, S0, 96), sp(S0, 192), wsp(fcwt), wsp(fcb),
                  sp(1, S1), sp(1, S1), sp(1, S1), sp(S1, C1), sp(1, S1),
                  sp(1, S1), sp(1, S1), wsp(wrelt), wsp(wst), wsp(bs)],
        out_specs=[sp(S0, C0), sp(S1, 384), sp(S1, 384)],
        out_shape=[jax.ShapeDtypeStruct((B, S0, C0), f32),
                   jax.ShapeDtypeStruct((B, S1, 384), f32),
                   jax.ShapeDtypeStruct((B, S1, 384), f32)],
        interpret=interpret,
    )(gmh, off0, fcwt, fcb, p2x, p2y, p2z, s1p, p1x, p1y, p1z, wrelt, wst, bs)


def _gru1_body(gm1, off1, x1, wxzt, wxrt, fcxt, fcst, fcb, s1n):
    o = gm1[0] + off1[0]
    x = x1[0]
    zin = o[:, :C1] + jnp.dot(x, wxzt[...], preferred_element_type=jnp.float32)
    rin = o[:, C1:2 * C1] + jnp.dot(x, wxrt[...],
                                    preferred_element_type=jnp.float32)
    so = o[:, 2 * C1:]
    z = jax.nn.sigmoid(zin)
    r = jax.nn.sigmoid(rin)
    sn = jnp.tanh(jnp.dot(x, fcxt[...], preferred_element_type=jnp.float32)
                  + jnp.dot(r * so, fcst[...],
                            preferred_element_type=jnp.float32) + fcb[...])
    s1n[0] = z * so + (1 - z) * sn


def _gru1(gm1, off1, x1, wxzt, wxrt, fcxt, fcst, fcb, interpret=False):
    f32 = jnp.float32
    sp = lambda *shp: pl.BlockSpec((1,) + shp, lambda b: (b,) + (0,) * len(shp))
    wsp = lambda a: pl.BlockSpec(a.shape, lambda b: (0,) * a.ndim)
    return pl.pallas_call(
        _gru1_body,
        grid=(B,),
        in_specs=[sp(S1, 384), sp(S1, 384), sp(S1, C0),
                  wsp(wxzt), wsp(wxrt), wsp(fcxt), wsp(fcst), wsp(fcb)],
        out_specs=[sp(S1, C1)],
        out_shape=[jax.ShapeDtypeStruct((B, S1, C1), f32)],
        interpret=interpret,
    )(gm1, off1, x1, wxzt, wxrt, fcxt, fcst, fcb)[0]


def _ln(x, g, b):
    m = jnp.mean(x, axis=-1, keepdims=True)
    v = jnp.mean((x - m) ** 2, axis=-1, keepdims=True)
    return (x - m) / jnp.sqrt(v + 1e-5) * g + b


def _tail_body(feats, posx, posy, posz, poswt, posb,
               q0, o0w, o0b, f01, f01b, f02, f02b, l01g, l01b, l02g, l02b,
               q1, o1w, o1b, f11, f11b, f12, f12b, l11g, l11b, l12g, l12b,
               hlg, hlb, hf1t, hf1b, hf2t, hf2b, out):
    x = _outer3(posx[0, 0], posy[0, 0], posz[0, 0], poswt[...]) + posb[...] + feats[0]
    layers = [(q0, o0w, o0b, f01, f01b, f02, f02b, l01g, l01b, l02g, l02b),
              (q1, o1w, o1b, f11, f11b, f12, f12b, l11g, l11b, l12g, l12b)]
    nt = T * S1
    for (qw, ow, ob, w1, b1, w2, b2, g1, bb1, g2, bb2) in layers:
        h = _ln(x, g1[...], bb1[...])
        qkv = jnp.dot(h, qw[...], preferred_element_type=jnp.float32)
        outs = []
        for hd in range(HEADS):
            q = qkv[:, hd * DIM_HEAD:(hd + 1) * DIM_HEAD]
            kk = qkv[:, 128 + hd * DIM_HEAD:128 + (hd + 1) * DIM_HEAD]
            v = qkv[:, 256 + hd * DIM_HEAD:256 + (hd + 1) * DIM_HEAD]
            att = jnp.dot(q, kk.T, preferred_element_type=jnp.float32)
            att = att / jnp.sqrt(jnp.float32(DIM_HEAD))
            att = att - jnp.max(att, axis=-1, keepdims=True)
            att = jnp.exp(att)
            att = att / jnp.sum(att, axis=-1, keepdims=True)
            outs.append(jnp.dot(att, v, preferred_element_type=jnp.float32))
        o = jnp.concatenate(outs, axis=1)
        x = x + jnp.dot(o, ow[...], preferred_element_type=jnp.float32) + ob[...]
        h = _ln(x, g2[...], bb2[...])
        h = jax.nn.gelu(jnp.dot(h, w1[...], preferred_element_type=jnp.float32)
                        + b1[...])
        x = x + jnp.dot(h, w2[...], preferred_element_type=jnp.float32) + b2[...]
    x = jax.nn.relu(x)
    e = jnp.max(x, axis=0, keepdims=True)
    e = _ln(e, hlg[...], hlb[...])
    e = jnp.dot(e, hf1t[...], preferred_element_type=jnp.float32) + hf1b[...]
    e = jnp.dot(e, hf2t[...], preferred_element_type=jnp.float32) + hf2b[...]
    out[0] = e


def _tail(feats, posx, posy, posz, args, interpret=False):
    f32 = jnp.float32
    sp = lambda *shp: pl.BlockSpec((1,) + shp, lambda b: (b,) + (0,) * len(shp))
    wsp = lambda a: pl.BlockSpec(a.shape, lambda b: (0,) * a.ndim)
    nt = T * S1
    posx, posy, posz = posx[:, None], posy[:, None], posz[:, None]
    return pl.pallas_call(
        _tail_body,
        grid=(B,),
        in_specs=[sp(nt, C1), sp(1, nt), sp(1, nt), sp(1, nt)]
        + [wsp(a) for a in args],
        out_specs=[sp(1, 40)],
        out_shape=[jax.ShapeDtypeStruct((B, 1, 40), f32)],
        interpret=interpret,
    )(feats, posx, posy, posz, *args)[0][:, 0]


def kernel(points, g0_z_W, g0_z_b, g0_r_W, g0_r_b, g0_s_W, g0_s_b, g0_fc_W,
           g0_fc_b, g1_z_W, g1_z_b, g1_r_W, g1_r_b, g1_s_W, g1_s_b, g1_fc_W,
           g1_fc_b, pos_W, pos_b, t0_qkv_W, t0_out_W, t0_out_b, t0_ff1_W,
           t0_ff1_b, t0_ff2_W, t0_ff2_b, t0_ln1_g, t0_ln1_b, t0_ln2_g,
           t0_ln2_b, t1_qkv_W, t1_out_W, t1_out_b, t1_ff1_W, t1_ff1_b,
           t1_ff2_W, t1_ff2_b, t1_ln1_g, t1_ln1_b, t1_ln2_g, t1_ln2_b,
           head_fc1_W, head_fc1_b, head_fc2_W, head_fc2_b, head_ln_g,
           head_ln_b):
    f32 = jnp.float32
    r2 = lambda a: a.reshape(1, -1)
    g0_wrelt = jnp.concatenate([g0_z_W[:, :3], g0_r_W[:, :3], g0_s_W[:, :3]], 0).T
    g0_wst = jnp.concatenate([g0_z_W[:, 3:3 + C0], g0_r_W[:, 3:3 + C0],
                              g0_s_W[:, 3:3 + C0]], 0).T
    g0_bs = r2(jnp.concatenate([g0_z_b, g0_r_b, g0_s_b], 0))
    g1_wrelt = jnp.concatenate([g1_z_W[:, :3], g1_r_W[:, :3], g1_s_W[:, :3]], 0).T
    g1_wst = jnp.concatenate([g1_z_W[:, 3:3 + C1], g1_r_W[:, 3:3 + C1],
                              g1_s_W[:, 3:3 + C1]], 0).T
    g1_bs = r2(jnp.concatenate([g1_z_b, g1_r_b, g1_s_b], 0))

    px = points[..., 0].reshape(G, N)
    py = points[..., 1].reshape(G, N)
    pz = points[..., 2].reshape(G, N)
    x0, y0, z0, x1, y1, z1 = _fps(px, py, pz)
    x0 = x0.reshape(B, T, S0); y0 = y0.reshape(B, T, S0); z0 = z0.reshape(B, T, S0)
    x1 = x1.reshape(B, T, S1); y1 = y1.reshape(B, T, S1); z1 = z1.reshape(B, T, S1)

    s0_state = jnp.zeros((B, S0, C0), f32)
    s1_state = jnp.zeros((B, S1, C1), f32)
    zplane1 = jnp.zeros((B, S1), f32)
    feats = []
    for t in range(T):
        tp = max(t - 1, 0)
        hh, off0 = _h0(x0[:, tp], y0[:, tp], z0[:, tp], s0_state,
                       x0[:, t], y0[:, t], z0[:, t],
                       g0_wrelt, g0_wst, g0_bs)
        gmh = _sc_g0(x0[:, t], y0[:, t], z0[:, t],
                     x0[:, tp], y0[:, tp], z0[:, tp],
                     hh.reshape(2 * B, S0 * 96)).reshape(B, 2, S0, 96)
        if t == 0:
            p2x1, p2y1, p2z1 = zplane1, zplane1, zplane1
        else:
            p2x1, p2y1, p2z1 = x1[:, t - 1], y1[:, t - 1], z1[:, t - 1]
        s0_state, h1, off1 = _gru0(gmh, off0, g0_fc_W.T, r2(g0_fc_b),
                                   p2x1, p2y1, p2z1, s1_state,
                                   x1[:, t], y1[:, t], z1[:, t],
                                   g1_wrelt, g1_wst, g1_bs)
        x1f = _sc_grp(x1[:, t], y1[:, t], z1[:, t],
                      x0[:, t], y0[:, t], z0[:, t],
                      s0_state.reshape(B, S0 * C0))
        gm1 = _sc_g1(x1[:, t], y1[:, t], z1[:, t], p2x1, p2y1, p2z1,
                     h1.reshape(B, S1 * 384))
        s1_state = _gru1(gm1, off1, x1f, g1_z_W[:, 3 + C1:].T,
                         g1_r_W[:, 3 + C1:].T, g1_fc_W[:, :C0].T,
                         g1_fc_W[:, C0:].T, r2(g1_fc_b))
        feats.append(s1_state)

    fa = jnp.stack(feats, axis=1)
    fa = fa.transpose(0, 1, 3, 2).reshape(B, T * S1, C1)
    posx = x1.reshape(B, T * S1)
    posy = y1.reshape(B, T * S1)
    posz = z1.reshape(B, T * S1)
    targs = [pos_W.T, r2(pos_b),
             t0_qkv_W, t0_out_W, r2(t0_out_b), t0_ff1_W, r2(t0_ff1_b),
             t0_ff2_W, r2(t0_ff2_b), r2(t0_ln1_g), r2(t0_ln1_b),
             r2(t0_ln2_g), r2(t0_ln2_b),
             t1_qkv_W, t1_out_W, r2(t1_out_b), t1_ff1_W, r2(t1_ff1_b),
             t1_ff2_W, r2(t1_ff2_b), r2(t1_ln1_g), r2(t1_ln1_b),
             r2(t1_ln2_g), r2(t1_ln2_b),
             r2(head_ln_g), r2(head_ln_b), head_fc1_W.T, r2(head_fc1_b),
             head_fc2_W.T, r2(head_fc2_b)]
    out = _tail(fa, posx, posy, posz, targs)
    return out.reshape(T, -1)


# speedup vs baseline: 22.8744x; 1.2364x over previous
import functools

import jax
import jax.numpy as jnp
from jax import lax
from jax.experimental import pallas as pl
from jax.experimental.pallas import tpu as pltpu
from jax.experimental.pallas import tpu_sc as plsc

B, T, N = 2, 4, 4096
S0, S1 = 1024, 256
C0, C1 = 64, 128
K0, KG, K1 = 64, 16, 48
R0SQ, RGSQ, R1SQ = 0.25, 0.25, 1.0
HEADS, DIM_HEAD, DEPTH, MLP_DIM = 4, 32, 2, 256
G = B * T
NEG = -3.4e38


def _fps_body(px, py, pz, x0, y0, z0, x1, y1, z1):
    def run_level(X, Y, Z, n, npt):
        iota = lax.broadcasted_iota(jnp.int32, (G, n), 1)
        iotap = lax.broadcasted_iota(jnp.int32, (G, npt), 1)

        def step(i, carry):
            dist, far, ax, ay, az = carry
            oh = (iota == far).astype(jnp.float32)
            cx = jnp.sum(X * oh, 1, keepdims=True)
            cy = jnp.sum(Y * oh, 1, keepdims=True)
            cz = jnp.sum(Z * oh, 1, keepdims=True)
            sel = (iotap == i).astype(jnp.float32)
            ax = ax + cx * sel
            ay = ay + cy * sel
            az = az + cz * sel
            dx = X - cx
            dy = Y - cy
            dz = Z - cz
            d = (dx * dx + dy * dy) + dz * dz
            dist = jnp.minimum(dist, d)
            m = jnp.max(dist, 1, keepdims=True)
            far = jnp.min(jnp.where(dist == m, iota, n), 1, keepdims=True)
            return dist, far, ax, ay, az

        zer = jnp.zeros((G, npt), jnp.float32)
        _, _, ax, ay, az = lax.fori_loop(
            0, npt, step,
            (jnp.full((G, n), 1e10, jnp.float32),
             jnp.zeros((G, 1), jnp.int32), zer, zer, zer))
        return ax, ay, az

    ax0, ay0, az0 = run_level(px[...], py[...], pz[...], N, S0)
    x0[...] = ax0
    y0[...] = ay0
    z0[...] = az0
    ax1, ay1, az1 = run_level(ax0, ay0, az0, S0, S1)
    x1[...] = ax1
    y1[...] = ay1
    z1[...] = az1


def _fps(px, py, pz, interpret=False):
    f32 = jnp.float32
    return pl.pallas_call(
        _fps_body,
        out_shape=[jax.ShapeDtypeStruct((G, S0), f32)] * 3
        + [jax.ShapeDtypeStruct((G, S1), f32)] * 3,
        interpret=interpret,
    )(px, py, pz)


def _take(v, idxvec):
    dn = lax.GatherDimensionNumbers(
        offset_dims=(), collapsed_slice_dims=(0,), start_index_map=(0,))
    return lax.gather(v, idxvec[:, None], dn, (1,),
                      mode=lax.GatherScatterMode.PROMISE_IN_BOUNDS)


def _make_sc_gm(nplanes, qsplit, s, ns, k, ch, r2, ppb):
    qpw = s // qsplit
    nch = ch // 16
    kg = k // 16
    nsc = ns // 16
    mesh = plsc.VectorSubcoreMesh(core_axis_name="c", subcore_axis_name="s",
                                  num_cores=2, num_subcores=16)

    def body(qx_h, qy_h, qz_h, sx_h, sy_h, sz_h, h_h, o_h,
             sxv, syv, szv, qxv, qyv, qzv, hv, ibuf, outv):
        wid = lax.axis_index("s") * 2 + lax.axis_index("c")
        plane = wid // qsplit
        qc = lax.rem(wid, qsplit)
        b = plane // ppb
        pltpu.sync_copy(sx_h.at[b], sxv)
        pltpu.sync_copy(sy_h.at[b], syv)
        pltpu.sync_copy(sz_h.at[b], szv)
        qoff = pl.multiple_of(qc * qpw, 16)
        pltpu.sync_copy(qx_h.at[b, pl.ds(qoff, qpw)], qxv)
        pltpu.sync_copy(qy_h.at[b, pl.ds(qoff, qpw)], qyv)
        pltpu.sync_copy(qz_h.at[b, pl.ds(qoff, qpw)], qzv)
        pltpu.sync_copy(h_h.at[plane], hv)
        iota16 = lax.iota(jnp.int32, 16)
        zeros16 = iota16 * 0

        def per_query(q, carry):
            qg = pl.multiple_of((q // 16) * 16, 16)
            lsel = zeros16 + lax.rem(q, 16)
            qxs = _take(qxv[pl.ds(qg, 16)], lsel)
            qys = _take(qyv[pl.ds(qg, 16)], lsel)
            qzs = _take(qzv[pl.ds(qg, 16)], lsel)
            ibuf[pl.ds(0, 16)] = zeros16

            def scan_step(cnt, ci):
                co = pl.multiple_of(ci * 16, 16)
                dx = sxv[pl.ds(co, 16)] - qxs
                dy = syv[pl.ds(co, 16)] - qys
                dz = szv[pl.ds(co, 16)] - qzs
                d = (dx * dx + dy * dy) + dz * dz
                m = d <= r2
                inc = plsc.cumsum(m.astype(jnp.int32))
                eff = m & (inc <= (k - cnt))
                plsc.store_scatter(ibuf, [cnt + inc - 1], iota16 + ci * 16,
                                   mask=eff)
                return cnt + jnp.sum(jnp.where(eff, 1, 0))

            def scan_plain(ci, cnt):
                return scan_step(cnt, ci)

            p1 = min(nsc, max(k // 4, 4))
            cnt = lax.fori_loop(0, p1, scan_plain, 0)
            if p1 < nsc:
                cnt = lax.cond(
                    cnt < k,
                    lambda c: lax.fori_loop(p1, nsc, scan_plain, c),
                    lambda c: c, cnt)
            cnt_v = zeros16 + cnt
            j0 = _take(ibuf[pl.ds(0, 16)], zeros16)
            colv = [iota16 + cc * 16 for cc in range(nch)]

            def ggroup(g, acc):
                go = pl.multiple_of(g * 16, 16)
                jv = ibuf[pl.ds(go, 16)]
                valid = (iota16 + g * 16) < cnt_v
                jv = jnp.where(valid, jv, j0)
                bases = jv * ch
                for l in range(16):
                    bsplat = _take(bases, zeros16 + l)
                    acc = tuple(
                        jnp.maximum(acc[cc],
                                    plsc.load_gather(hv, [bsplat + colv[cc]]))
                        for cc in range(nch))
                return acc

            acc = lax.fori_loop(0, kg, ggroup,
                                tuple(zeros16.astype(jnp.float32) + NEG
                                      for _ in range(nch)))
            for cc in range(nch):
                outv[q, pl.ds(cc * 16, 16)] = acc[cc]
            return carry

        lax.fori_loop(0, qpw, per_query, 0)
        pltpu.sync_copy(outv, o_h.at[plane, pl.ds(qoff, qpw)])

    f32 = jnp.float32
    return pl.kernel(
        body,
        out_type=jax.ShapeDtypeStruct((nplanes, s, ch), f32),
        mesh=mesh,
        compiler_params=pltpu.CompilerParams(needs_layout_passes=False),
        scratch_types=[
            pltpu.VMEM((ns,), f32), pltpu.VMEM((ns,), f32),
            pltpu.VMEM((ns,), f32),
            pltpu.VMEM((qpw,), f32), pltpu.VMEM((qpw,), f32),
            pltpu.VMEM((qpw,), f32),
            pltpu.VMEM((ns * ch,), f32),
            pltpu.VMEM((k + 16,), jnp.int32),
            pltpu.VMEM((qpw, ch), f32),
        ],
    )


_sc_gm_cached = functools.lru_cache(None)(_make_sc_gm)


def _sc_g0(*a):
    return _sc_gm_cached(4, 8, S0, S0, K0, 96, R0SQ, 2)(*a)


def _sc_grp(*a):
    return _sc_gm_cached(2, 16, S1, S0, KG, C0, RGSQ, 1)(*a)


def _sc_g1(*a):
    return _sc_gm_cached(2, 16, S1, S1, K1, 384, R1SQ, 1)(*a)


def _outer3(x, y, z, wt):
    return (x[:, None] * wt[0][None, :] + y[:, None] * wt[1][None, :]
            + z[:, None] * wt[2][None, :])


def _h0_body(p2x, p2y, p2z, s2, p1x, p1y, p1z, wrelt, wst, bs, hh, off):
    w = wrelt[...]
    h = _outer3(p2x[0, 0], p2y[0, 0], p2z[0, 0], w) + jnp.dot(
        s2[0], wst[...], preferred_element_type=jnp.float32)
    hh[0, 0] = h[:, :96]
    hh[0, 1] = h[:, 96:]
    off[0] = bs[...] - _outer3(p1x[0, 0], p1y[0, 0], p1z[0, 0], w)


def _h0(p2x, p2y, p2z, s2, p1x, p1y, p1z, wrelt, wst, bs, interpret=False):
    f32 = jnp.float32
    sp = lambda *shp: pl.BlockSpec((1,) + shp, lambda b: (b,) + (0,) * len(shp))
    wsp = lambda a: pl.BlockSpec(a.shape, lambda b: (0,) * a.ndim)
    p2x, p2y, p2z = p2x[:, None], p2y[:, None], p2z[:, None]
    p1x, p1y, p1z = p1x[:, None], p1y[:, None], p1z[:, None]
    return pl.pallas_call(
        _h0_body,
        grid=(B,),
        in_specs=[sp(1, S0), sp(1, S0), sp(1, S0), sp(S0, C0), sp(1, S0),
                  sp(1, S0), sp(1, S0), wsp(wrelt), wsp(wst), wsp(bs)],
        out_specs=[sp(2, S0, 96), sp(S0, 192)],
        out_shape=[jax.ShapeDtypeStruct((B, 2, S0, 96), f32),
                   jax.ShapeDtypeStruct((B, S0, 192), f32)],
        interpret=interpret,
    )(p2x, p2y, p2z, s2, p1x, p1y, p1z, wrelt, wst, bs)


def _gru0_body(gmh, off0, fcwt, fcb, p2x, p2y, p2z, s1p, p1x, p1y, p1z,
               wrelt, wst, bs, s0n, h1, off1):
    o = jnp.concatenate([gmh[0, 0], gmh[0, 1]], axis=1) + off0[0]
    z = jax.nn.sigmoid(o[:, :C0])
    r = jax.nn.sigmoid(o[:, C0:2 * C0])
    so = o[:, 2 * C0:]
    sn = jnp.tanh(jnp.dot(r * so, fcwt[...],
                          preferred_element_type=jnp.float32) + fcb[...])
    s0n[0] = z * so + (1 - z) * sn
    w = wrelt[...]
    h1[0] = _outer3(p2x[0, 0], p2y[0, 0], p2z[0, 0], w) + jnp.dot(
        s1p[0], wst[...], preferred_element_type=jnp.float32)
    off1[0] = bs[...] - _outer3(p1x[0, 0], p1y[0, 0], p1z[0, 0], w)


def _gru0(gmh, off0, fcwt, fcb, p2x, p2y, p2z, s1p, p1x, p1y, p1z,
          wrelt, wst, bs, interpret=False):
    f32 = jnp.float32
    sp = lambda *shp: pl.BlockSpec((1,) + shp, lambda b: (b,) + (0,) * len(shp))
    wsp = lambda a: pl.BlockSpec(a.shape, lambda b: (0,) * a.ndim)
    p2x, p2y, p2z = p2x[:, None], p2y[:, None], p2z[:, None]
    p1x, p1y, p1z = p1x[:, None], p1y[:, None], p1z[:, None]
    return pl.pallas_call(
        _gru0_body,
        grid=(B,),
        in_specs=[sp(2, S0, 96), sp(S0, 192), wsp(fcwt), wsp(fcb),
                  sp(1, S1), sp(1, S1), sp(1, S1), sp(S1, C1), sp(1, S1),
                  sp(1, S1), sp(1, S1), wsp(wrelt), wsp(wst), wsp(bs)],
        out_specs=[sp(S0, C0), sp(S1, 384), sp(S1, 384)],
        out_shape=[jax.ShapeDtypeStruct((B, S0, C0), f32),
                   jax.ShapeDtypeStruct((B, S1, 384), f32),
                   jax.ShapeDtypeStruct((B, S1, 384), f32)],
        interpret=interpret,
    )(gmh, off0, fcwt, fcb, p2x, p2y, p2z, s1p, p1x, p1y, p1z, wrelt, wst, bs)


def _gru1_body(gm1, off1, x1, wxzt, wxrt, fcxt, fcst, fcb, s1n):
    o = gm1[0] + off1[0]
    x = x1[0]
    zin = o[:, :C1] + jnp.dot(x, wxzt[...], preferred_element_type=jnp.float32)
    rin = o[:, C1:2 * C1] + jnp.dot(x, wxrt[...],
                                    preferred_element_type=jnp.float32)
    so = o[:, 2 * C1:]
    z = jax.nn.sigmoid(zin)
    r = jax.nn.sigmoid(rin)
    sn = jnp.tanh(jnp.dot(x, fcxt[...], preferred_element_type=jnp.float32)
                  + jnp.dot(r * so, fcst[...],
                            preferred_element_type=jnp.float32) + fcb[...])
    s1n[0] = z * so + (1 - z) * sn


def _gru1(gm1, off1, x1, wxzt, wxrt, fcxt, fcst, fcb, interpret=False):
    f32 = jnp.float32
    sp = lambda *shp: pl.BlockSpec((1,) + shp, lambda b: (b,) + (0,) * len(shp))
    wsp = lambda a: pl.BlockSpec(a.shape, lambda b: (0,) * a.ndim)
    return pl.pallas_call(
        _gru1_body,
        grid=(B,),
        in_specs=[sp(S1, 384), sp(S1, 384), sp(S1, C0),
                  wsp(wxzt), wsp(wxrt), wsp(fcxt), wsp(fcst), wsp(fcb)],
        out_specs=[sp(S1, C1)],
        out_shape=[jax.ShapeDtypeStruct((B, S1, C1), f32)],
        interpret=interpret,
    )(gm1, off1, x1, wxzt, wxrt, fcxt, fcst, fcb)[0]


def _ln(x, g, b):
    m = jnp.mean(x, axis=-1, keepdims=True)
    v = jnp.mean((x - m) ** 2, axis=-1, keepdims=True)
    return (x - m) / jnp.sqrt(v + 1e-5) * g + b


def _tail_body(feats, posx, posy, posz, poswt, posb,
               q0, o0w, o0b, f01, f01b, f02, f02b, l01g, l01b, l02g, l02b,
               q1, o1w, o1b, f11, f11b, f12, f12b, l11g, l11b, l12g, l12b,
               hlg, hlb, hf1t, hf1b, hf2t, hf2b, out):
    x = _outer3(posx[0, 0], posy[0, 0], posz[0, 0], poswt[...]) + posb[...] + feats[0]
    layers = [(q0, o0w, o0b, f01, f01b, f02, f02b, l01g, l01b, l02g, l02b),
              (q1, o1w, o1b, f11, f11b, f12, f12b, l11g, l11b, l12g, l12b)]
    nt = T * S1
    for (qw, ow, ob, w1, b1, w2, b2, g1, bb1, g2, bb2) in layers:
        h = _ln(x, g1[...], bb1[...])
        qkv = jnp.dot(h, qw[...], preferred_element_type=jnp.float32)
        outs = []
        for hd in range(HEADS):
            q = qkv[:, hd * DIM_HEAD:(hd + 1) * DIM_HEAD]
            kk = qkv[:, 128 + hd * DIM_HEAD:128 + (hd + 1) * DIM_HEAD]
            v = qkv[:, 256 + hd * DIM_HEAD:256 + (hd + 1) * DIM_HEAD]
            att = jnp.dot(q, kk.T, preferred_element_type=jnp.float32)
            att = att / jnp.sqrt(jnp.float32(DIM_HEAD))
            att = att - jnp.max(att, axis=-1, keepdims=True)
            att = jnp.exp(att)
            att = att / jnp.sum(att, axis=-1, keepdims=True)
            outs.append(jnp.dot(att, v, preferred_element_type=jnp.float32))
        o = jnp.concatenate(outs, axis=1)
        x = x + jnp.dot(o, ow[...], preferred_element_type=jnp.float32) + ob[...]
        h = _ln(x, g2[...], bb2[...])
        h = jax.nn.gelu(jnp.dot(h, w1[...], preferred_element_type=jnp.float32)
                        + b1[...])
        x = x + jnp.dot(h, w2[...], preferred_element_type=jnp.float32) + b2[...]
    x = jax.nn.relu(x)
    e = jnp.max(x, axis=0, keepdims=True)
    e = _ln(e, hlg[...], hlb[...])
    e = jnp.dot(e, hf1t[...], preferred_element_type=jnp.float32) + hf1b[...]
    e = jnp.dot(e, hf2t[...], preferred_element_type=jnp.float32) + hf2b[...]
    out[0] = e


def _tail(feats, posx, posy, posz, args, interpret=False):
    f32 = jnp.float32
    sp = lambda *shp: pl.BlockSpec((1,) + shp, lambda b: (b,) + (0,) * len(shp))
    wsp = lambda a: pl.BlockSpec(a.shape, lambda b: (0,) * a.ndim)
    nt = T * S1
    posx, posy, posz = posx[:, None], posy[:, None], posz[:, None]
    return pl.pallas_call(
        _tail_body,
        grid=(B,),
        in_specs=[sp(nt, C1), sp(1, nt), sp(1, nt), sp(1, nt)]
        + [wsp(a) for a in args],
        out_specs=[sp(1, 40)],
        out_shape=[jax.ShapeDtypeStruct((B, 1, 40), f32)],
        interpret=interpret,
    )(feats, posx, posy, posz, *args)[0][:, 0]


def kernel(points, g0_z_W, g0_z_b, g0_r_W, g0_r_b, g0_s_W, g0_s_b, g0_fc_W,
           g0_fc_b, g1_z_W, g1_z_b, g1_r_W, g1_r_b, g1_s_W, g1_s_b, g1_fc_W,
           g1_fc_b, pos_W, pos_b, t0_qkv_W, t0_out_W, t0_out_b, t0_ff1_W,
           t0_ff1_b, t0_ff2_W, t0_ff2_b, t0_ln1_g, t0_ln1_b, t0_ln2_g,
           t0_ln2_b, t1_qkv_W, t1_out_W, t1_out_b, t1_ff1_W, t1_ff1_b,
           t1_ff2_W, t1_ff2_b, t1_ln1_g, t1_ln1_b, t1_ln2_g, t1_ln2_b,
           head_fc1_W, head_fc1_b, head_fc2_W, head_fc2_b, head_ln_g,
           head_ln_b):
    f32 = jnp.float32
    r2 = lambda a: a.reshape(1, -1)
    g0_wrelt = jnp.concatenate([g0_z_W[:, :3], g0_r_W[:, :3], g0_s_W[:, :3]], 0).T
    g0_wst = jnp.concatenate([g0_z_W[:, 3:3 + C0], g0_r_W[:, 3:3 + C0],
                              g0_s_W[:, 3:3 + C0]], 0).T
    g0_bs = r2(jnp.concatenate([g0_z_b, g0_r_b, g0_s_b], 0))
    g1_wrelt = jnp.concatenate([g1_z_W[:, :3], g1_r_W[:, :3], g1_s_W[:, :3]], 0).T
    g1_wst = jnp.concatenate([g1_z_W[:, 3:3 + C1], g1_r_W[:, 3:3 + C1],
                              g1_s_W[:, 3:3 + C1]], 0).T
    g1_bs = r2(jnp.concatenate([g1_z_b, g1_r_b, g1_s_b], 0))

    px = points[..., 0].reshape(G, N)
    py = points[..., 1].reshape(G, N)
    pz = points[..., 2].reshape(G, N)
    x0, y0, z0, x1, y1, z1 = _fps(px, py, pz)
    x0 = x0.reshape(B, T, S0); y0 = y0.reshape(B, T, S0); z0 = z0.reshape(B, T, S0)
    x1 = x1.reshape(B, T, S1); y1 = y1.reshape(B, T, S1); z1 = z1.reshape(B, T, S1)

    s0_state = jnp.zeros((B, S0, C0), f32)
    s1_state = jnp.zeros((B, S1, C1), f32)
    zplane1 = jnp.zeros((B, S1), f32)
    feats = []
    for t in range(T):
        tp = max(t - 1, 0)
        hh, off0 = _h0(x0[:, tp], y0[:, tp], z0[:, tp], s0_state,
                       x0[:, t], y0[:, t], z0[:, t],
                       g0_wrelt, g0_wst, g0_bs)
        gmh = _sc_g0(x0[:, t], y0[:, t], z0[:, t],
                     x0[:, tp], y0[:, tp], z0[:, tp],
                     hh.reshape(2 * B, S0 * 96)).reshape(B, 2, S0, 96)
        if t == 0:
            p2x1, p2y1, p2z1 = zplane1, zplane1, zplane1
        else:
            p2x1, p2y1, p2z1 = x1[:, t - 1], y1[:, t - 1], z1[:, t - 1]
        s0_state, h1, off1 = _gru0(gmh, off0, g0_fc_W.T, r2(g0_fc_b),
                                   p2x1, p2y1, p2z1, s1_state,
                                   x1[:, t], y1[:, t], z1[:, t],
                                   g1_wrelt, g1_wst, g1_bs)
        x1f = _sc_grp(x1[:, t], y1[:, t], z1[:, t],
                      x0[:, t], y0[:, t], z0[:, t],
                      s0_state.reshape(B, S0 * C0))
        gm1 = _sc_g1(x1[:, t], y1[:, t], z1[:, t], p2x1, p2y1, p2z1,
                     h1.reshape(B, S1 * 384))
        s1_state = _gru1(gm1, off1, x1f, g1_z_W[:, 3 + C1:].T,
                         g1_r_W[:, 3 + C1:].T, g1_fc_W[:, :C0].T,
                         g1_fc_W[:, C0:].T, r2(g1_fc_b))
        feats.append(s1_state)

    fa = jnp.stack(feats, axis=1)
    fa = fa.transpose(0, 1, 3, 2).reshape(B, T * S1, C1)
    posx = x1.reshape(B, T * S1)
    posy = y1.reshape(B, T * S1)
    posz = z1.reshape(B, T * S1)
    targs = [pos_W.T, r2(pos_b),
             t0_qkv_W, t0_out_W, r2(t0_out_b), t0_ff1_W, r2(t0_ff1_b),
             t0_ff2_W, r2(t0_ff2_b), r2(t0_ln1_g), r2(t0_ln1_b),
             r2(t0_ln2_g), r2(t0_ln2_b),
             t1_qkv_W, t1_out_W, r2(t1_out_b), t1_ff1_W, r2(t1_ff1_b),
             t1_ff2_W, r2(t1_ff2_b), r2(t1_ln1_g), r2(t1_ln1_b),
             r2(t1_ln2_g), r2(t1_ln2_b),
             r2(head_ln_g), r2(head_ln_b), head_fc1_W.T, r2(head_fc1_b),
             head_fc2_W.T, r2(head_fc2_b)]
    out = _tail(fa, posx, posy, posz, targs)
    return out.reshape(T, -1)
